# baseline (speedup 1.0000x reference)
import functools

import jax
import jax.numpy as jnp
from jax import lax
from jax.experimental import pallas as pl
from jax.experimental.pallas import tpu as pltpu
from jax.experimental.pallas import tpu_sc as plsc

_DIM = 64
_NC = 2
_NS = 16
_NW = _NC * _NS
_CHUNK = 128


def _make_gather(n_rows):
    b_per_w = n_rows // _NW
    n_chunks = b_per_w // _CHUNK
    mesh = plsc.VectorSubcoreMesh(core_axis_name="c", subcore_axis_name="s")

    @functools.partial(
        pl.kernel,
        mesh=mesh,
        out_type=jax.ShapeDtypeStruct((n_rows, _DIM), jnp.float32),
        scratch_types=[
            pltpu.VMEM((n_chunks, _CHUNK), jnp.int32),
            pltpu.VMEM((_CHUNK, _DIM), jnp.float32),
            pltpu.SemaphoreType.DMA,
        ],
        compiler_params=pltpu.CompilerParams(use_tc_tiling_on_sc=False),
    )
    def gather_kernel(idx_hbm, table_hbm, out_hbm, idx_v, rows_v, sem):
        wid = lax.axis_index("s") * _NC + lax.axis_index("c")
        base = wid * b_per_w
        pltpu.sync_copy(idx_hbm.at[wid], idx_v)

        def body(j, carry):
            pltpu.async_copy(table_hbm.at[idx_v.at[j]], rows_v, sem).wait()
            pltpu.sync_copy(rows_v, out_hbm.at[pl.ds(base + j * _CHUNK, _CHUNK)])
            return carry

        lax.fori_loop(0, n_chunks, body, 0)

    return gather_kernel


def kernel(x, table):
    b, l = x.shape
    n = b * l
    idx = x.reshape(_NW, n // (_NW * _CHUNK), _CHUNK)
    out = _make_gather(n)(idx, table)
    return out.reshape(b, l, _DIM)


# speedup vs baseline: 1.0911x; 1.0911x over previous
import functools

import jax
import jax.numpy as jnp
from jax import lax
from jax.experimental import pallas as pl
from jax.experimental.pallas import tpu as pltpu
from jax.experimental.pallas import tpu_sc as plsc

_DIM = 64
_NC = 2
_NS = 16
_NW = _NC * _NS
_CHUNK = 128


def _make_gather(n_rows):
    b_per_w = n_rows // _NW
    n_chunks = b_per_w // _CHUNK
    n_rounds = n_chunks // 2
    assert n_chunks % 2 == 0 and n_rounds >= 2
    mesh = plsc.VectorSubcoreMesh(core_axis_name="c", subcore_axis_name="s")

    @functools.partial(
        pl.kernel,
        mesh=mesh,
        out_type=jax.ShapeDtypeStruct((n_rows, _DIM), jnp.float32),
        scratch_types=[
            pltpu.VMEM((n_chunks, _CHUNK), jnp.int32),
            pltpu.VMEM((_CHUNK, _DIM), jnp.float32),
            pltpu.VMEM((_CHUNK, _DIM), jnp.float32),
            pltpu.SemaphoreType.DMA,
            pltpu.SemaphoreType.DMA,
        ],
        compiler_params=pltpu.CompilerParams(use_tc_tiling_on_sc=False),
    )
    def gather_kernel(idx_hbm, table_hbm, out_hbm, idx_v, rows0, rows1, s0, s1):
        rows = (rows0, rows1)
        sem = (s0, s1)
        wid = lax.axis_index("s") * _NC + lax.axis_index("c")
        base = wid * b_per_w
        pltpu.sync_copy(idx_hbm.at[wid], idx_v)

        def start_gather(g, b):
            pltpu.async_copy(table_hbm.at[idx_v.at[g]], rows[b], sem[b])

        def wait_gather(g, b):
            pltpu.make_async_copy(
                table_hbm.at[idx_v.at[g]], rows[b], sem[b]
            ).wait()

        def copy_out(g, b):
            pltpu.sync_copy(rows[b], out_hbm.at[pl.ds(base + g * _CHUNK, _CHUNK)])

        start_gather(0, 0)

        def round_body(r, carry):
            g0 = 2 * r
            for b in range(2):
                g = g0 + b
                start_gather(g + 1, 1 - b)
                wait_gather(g, b)
                copy_out(g, b)
            return carry

        lax.fori_loop(0, n_rounds - 1, round_body, 0)

        g0 = n_chunks - 2
        start_gather(g0 + 1, 1)
        wait_gather(g0, 0)
        copy_out(g0, 0)
        wait_gather(g0 + 1, 1)
        copy_out(g0 + 1, 1)

    return gather_kernel


def kernel(x, table):
    b, l = x.shape
    n = b * l
    idx = x.reshape(_NW, n // (_NW * _CHUNK), _CHUNK)
    out = _make_gather(n)(idx, table)
    return out.reshape(b, l, _DIM)


# speedup vs baseline: 1.1109x; 1.0182x over previous
import functools

import jax
import jax.numpy as jnp
from jax import lax
from jax.experimental import pallas as pl
from jax.experimental.pallas import tpu as pltpu
from jax.experimental.pallas import tpu_sc as plsc

_DIM = 64
_NC = 2
_NS = 16
_NW = _NC * _NS


def _make_gather(B, L):
    b_per_w = B // _NW
    mesh = plsc.VectorSubcoreMesh(core_axis_name="c", subcore_axis_name="s")

    @functools.partial(
        pl.kernel,
        mesh=mesh,
        out_type=jax.ShapeDtypeStruct((B, L, _DIM), jnp.float32),
        scratch_types=[
            pltpu.VMEM((b_per_w, L), jnp.int32),
            pltpu.VMEM((L, _DIM), jnp.float32),
            pltpu.VMEM((L, _DIM), jnp.float32),
            pltpu.SemaphoreType.DMA,
            pltpu.SemaphoreType.DMA,
        ],
        compiler_params=pltpu.CompilerParams(use_tc_tiling_on_sc=False),
    )
    def gather_kernel(x_hbm, table_hbm, out_hbm, idx_v, rows0, rows1, s0, s1):
        rows = (rows0, rows1)
        sem = (s0, s1)
        wid = lax.axis_index("s") * _NC + lax.axis_index("c")
        base = wid * b_per_w
        pltpu.sync_copy(x_hbm.at[pl.ds(base, b_per_w)], idx_v)

        def start_gather(g, b):
            pltpu.async_copy(table_hbm.at[idx_v.at[g]], rows[b], sem[b])

        def wait_gather(g, b):
            pltpu.make_async_copy(
                table_hbm.at[idx_v.at[g]], rows[b], sem[b]
            ).wait()

        def copy_out(g, b):
            pltpu.sync_copy(rows[b], out_hbm.at[base + g])

        start_gather(0, 0)

        def round_body(r, carry):
            g0 = 2 * r
            for b in range(2):
                g = g0 + b
                start_gather(g + 1, 1 - b)
                wait_gather(g, b)
                copy_out(g, b)
            return carry

        lax.fori_loop(0, b_per_w // 2 - 1, round_body, 0)

        g0 = b_per_w - 2
        start_gather(g0 + 1, 1)
        wait_gather(g0, 0)
        copy_out(g0, 0)
        wait_gather(g0 + 1, 1)
        copy_out(g0 + 1, 1)

    return gather_kernel


def kernel(x, table):
    b, l = x.shape
    return _make_gather(b, l)(x, table)


# speedup vs baseline: 1.4797x; 1.3320x over previous
import functools

import jax
import jax.numpy as jnp
from jax import lax
from jax.experimental import pallas as pl
from jax.experimental.pallas import tpu as pltpu
from jax.experimental.pallas import tpu_sc as plsc

_PD = 128
_NC = 2
_NS = 16
_NW = _NC * _NS


def _make_gather(B, L):
    b_per_w = B // _NW
    mesh = plsc.VectorSubcoreMesh(core_axis_name="c", subcore_axis_name="s")

    @functools.partial(
        pl.kernel,
        mesh=mesh,
        out_type=jax.ShapeDtypeStruct((B, L, _PD), jnp.float32),
        scratch_types=[
            pltpu.VMEM((b_per_w, L), jnp.int32),
            pltpu.VMEM((L, 64), jnp.float32),
            pltpu.VMEM((L, 64), jnp.float32),
            pltpu.SemaphoreType.DMA,
            pltpu.SemaphoreType.DMA,
        ],
        compiler_params=pltpu.CompilerParams(use_tc_tiling_on_sc=False),
    )
    def gather_kernel(x_hbm, table_hbm, out_hbm, idx_v, rows0, rows1, s0, s1):
        rows = (rows0, rows1)
        sem = (s0, s1)
        wid = lax.axis_index("s") * _NC + lax.axis_index("c")
        base = wid * b_per_w
        pltpu.sync_copy(x_hbm.at[pl.ds(base, b_per_w)], idx_v)

        def start_gather(g, b):
            pltpu.async_copy(
                table_hbm.at[idx_v.at[g]], rows[b], sem[b]
            )

        def wait_gather(g, b):
            pltpu.make_async_copy(
                table_hbm.at[idx_v.at[g]], rows[b], sem[b]
            ).wait()

        def copy_out(g, b):
            pltpu.sync_copy(rows[b], out_hbm.at[base + g, :, pl.ds(0, 64)])

        start_gather(0, 0)

        def round_body(r, carry):
            g0 = 2 * r
            for b in range(2):
                g = g0 + b
                start_gather(g + 1, 1 - b)
                wait_gather(g, b)
                copy_out(g, b)
            return carry

        lax.fori_loop(0, b_per_w // 2 - 1, round_body, 0)

        g0 = b_per_w - 2
        start_gather(g0 + 1, 1)
        wait_gather(g0, 0)
        copy_out(g0, 0)
        wait_gather(g0 + 1, 1)
        copy_out(g0 + 1, 1)

    return gather_kernel


def kernel(x, table):
    b, l = x.shape
    v, d = table.shape
    out = _make_gather(b, l)(x, table)
    return out[:, :, :d]


# speedup vs baseline: 1.4818x; 1.0014x over previous
import functools

import jax
import jax.numpy as jnp
from jax import lax
from jax.experimental import pallas as pl
from jax.experimental.pallas import tpu as pltpu
from jax.experimental.pallas import tpu_sc as plsc

_PD = 128
_NC = 2
_NS = 16
_NW = _NC * _NS


def _make_gather(B, L):
    b_per_w = B // _NW
    mesh = plsc.VectorSubcoreMesh(core_axis_name="c", subcore_axis_name="s")

    @functools.partial(
        pl.kernel,
        mesh=mesh,
        out_type=jax.ShapeDtypeStruct((B, L, _PD), jnp.float32),
        scratch_types=[
            pltpu.VMEM((b_per_w, L), jnp.int32),
        ]
        + [pltpu.VMEM((L, 64), jnp.float32)] * 4
        + [pltpu.SemaphoreType.DMA] * 4,
        compiler_params=pltpu.CompilerParams(use_tc_tiling_on_sc=False),
    )
    def gather_kernel(x_hbm, table_hbm, out_hbm, idx_v, *scr):
        rows = scr[:4]
        sem = scr[4:]
        wid = lax.axis_index("s") * _NC + lax.axis_index("c")
        base = wid * b_per_w
        pltpu.sync_copy(x_hbm.at[pl.ds(base, b_per_w)], idx_v)

        def start_gather(g, b):
            pltpu.async_copy(
                table_hbm.at[idx_v.at[g]], rows[b], sem[b]
            )

        def wait_gather(g, b):
            pltpu.make_async_copy(
                table_hbm.at[idx_v.at[g]], rows[b], sem[b]
            ).wait()

        def copy_out(g, b):
            pltpu.sync_copy(rows[b], out_hbm.at[base + g, :, pl.ds(0, 64)])

        for g in range(3):
            start_gather(g, g)

        def round_body(r, carry):
            g0 = 4 * r
            for b in range(4):
                g = g0 + b
                start_gather(g + 3, (b + 3) % 4)
                wait_gather(g, b)
                copy_out(g, b)
            return carry

        lax.fori_loop(0, b_per_w // 4 - 1, round_body, 0)

        g0 = b_per_w - 4
        start_gather(g0 + 3, 3)
        for b in range(4):
            g = g0 + b
            wait_gather(g, b)
            copy_out(g, b)

    return gather_kernel


def kernel(x, table):
    b, l = x.shape
    v, d = table.shape
    out = _make_gather(b, l)(x, table)
    return out[:, :, :d]
